# baseline (speedup 1.0000x reference)
import functools

import jax
import jax.numpy as jnp
from jax import lax
from jax.experimental import pallas as pl
from jax.experimental.pallas import tpu as pltpu
from jax.experimental.pallas import tpu_sc as plsc

NC = 2
NS = 16
NW = NC * NS
CHUNK = 128

_mesh = functools.partial(
    plsc.VectorSubcoreMesh, core_axis_name="c", subcore_axis_name="s"
)


def _worker_ids():
  c = lax.axis_index("c")
  s = lax.axis_index("s")
  return c, s, c * NS + s


def _row_partition(n):
  zr = 80
  rpt = -(-n // (NS * zr)) * zr
  last = n - rpt * (NS - 1)
  assert last > 0 and last % zr == 0
  return zr, rpt, last


def _make_deg_kernel(n, e_pad):
  nch = e_pad // (NW * CHUNK)
  zr, rpt, last = _row_partition(n)
  n_acc = rpt * NS
  dw = 128

  @functools.partial(
      pl.kernel,
      out_type=jax.ShapeDtypeStruct((NC, n, dw), jnp.float32),
      mesh=_mesh(),
      scratch_types=[
          pltpu.VMEM((nch, CHUNK), jnp.int32),
          pltpu.VMEM((CHUNK, dw), jnp.float32),
          pltpu.VMEM((zr, dw), jnp.float32),
          pltpu.VMEM_SHARED((n_acc, dw), jnp.float32),
          pltpu.SemaphoreType.DMA,
      ],
  )
  def deg_kernel(dst_hbm, out_hbm, idx_d, ones_v, zeros_v, acc, sem):
    c, s, w = _worker_ids()

    cbase = pl.multiple_of(w * nch, nch)
    pltpu.sync_copy(dst_hbm.at[pl.ds(cbase, nch)], idx_d)

    def fill_ones(i, _):
      for j in range(dw // 16):
        ones_v[i, pl.ds(j * 16, 16)] = jnp.full((16,), 1.0, jnp.float32)
      return _

    lax.fori_loop(0, CHUNK, fill_ones, None)

    def fill_zeros(i, _):
      for j in range(dw // 16):
        zeros_v[i, pl.ds(j * 16, 16)] = jnp.zeros((16,), jnp.float32)
      return _

    lax.fori_loop(0, zr, fill_zeros, None)

    base = pl.multiple_of(s * rpt, zr)
    nrows = jnp.where(s == NS - 1, last, rpt)

    def zero_b(b, _):
      pltpu.sync_copy(zeros_v, acc.at[pl.ds(pl.multiple_of(base + b * zr, zr), zr)])
      return _

    lax.fori_loop(0, nrows // zr, zero_b, None)
    plsc.subcore_barrier()

    def body(g, _):
      pltpu.sync_copy(ones_v, acc.at[idx_d.at[g]], add=True)
      return _

    lax.fori_loop(0, nch, body, None)
    plsc.subcore_barrier()

    @pl.when(s < NS - 1)
    def _():
      pltpu.sync_copy(acc.at[pl.ds(base, rpt)], out_hbm.at[c, pl.ds(base, rpt)])

    @pl.when(s == NS - 1)
    def _():
      pltpu.sync_copy(acc.at[pl.ds(base, last)], out_hbm.at[c, pl.ds(base, last)])

  return deg_kernel


def _make_prop_kernel(n, d, e_pad, num_h, depth=2, qc=None):
  nch = e_pad // (NW * CHUNK)
  qc0, qc1 = qc if qc is not None else (nch, nch)
  assert qc0 + qc1 == 2 * nch and qc0 % depth == 0 and qc1 % depth == 0
  zr, rpt, last = _row_partition(n)
  n_acc = rpt * NS
  zrows = 80

  @functools.partial(
      pl.kernel,
      out_type=[jax.ShapeDtypeStruct((NC, n, d), jnp.float32)] * num_h,
      mesh=_mesh(),
      scratch_types=[
          [pltpu.VMEM((CHUNK,), jnp.int32)] * depth,
          [pltpu.VMEM((CHUNK,), jnp.int32)] * depth,
          [pltpu.VMEM((CHUNK, d), jnp.float32)] * depth,
          pltpu.VMEM((zrows, d), jnp.float32),
          pltpu.VMEM((rpt // zrows, zrows), jnp.int32),
          pltpu.VMEM_SHARED((n_acc, d), jnp.float32),
          [pltpu.SemaphoreType.DMA] * depth,
          [pltpu.SemaphoreType.DMA] * depth,
          [pltpu.SemaphoreType.DMA] * depth,
          pltpu.SemaphoreType.DMA,
      ],
  )
  def prop_kernel(*args):
    h_refs = args[:num_h]
    src_hbm, dst_hbm = args[num_h], args[num_h + 1]
    out_refs = args[num_h + 2:num_h + 2 + num_h]
    (idx_s, idx_d, rows, zbuf, ziota, acc,
     gsem, ssem, dsem, zsem) = args[num_h + 2 + num_h:]
    c, s, w = _worker_ids()

    base = pl.multiple_of(s * rpt, zr)
    iota16 = lax.iota(jnp.int32, 16)

    def fill_zeros(i, _):
      for j in range(d // 16):
        zbuf[i, pl.ds(j * 16, 16)] = jnp.zeros((16,), jnp.float32)
      for j in range(zrows // 16):
        ziota[i % (rpt // zrows), pl.ds(j * 16, 16)] = (
            base + (i % (rpt // zrows)) * zrows + j * 16 + iota16)
      return _

    lax.fori_loop(0, max(zrows, rpt // zrows), fill_zeros, None)

    nrows = jnp.where(s == NS - 1, last, rpt)
    nq = jnp.where(c == 0, qc0, qc1)
    cbase = jnp.where(c == 0, s * qc0, NS * qc0 + s * qc1)

    def idx_off(g):
      return pl.multiple_of((cbase + g) * CHUNK, CHUNK)

    for h_hbm, out_hbm in zip(h_refs, out_refs):

      def zero_issue(k, _):
        pltpu.async_copy(zbuf, acc.at[ziota.at[k]], zsem)
        return _

      def zero_drain(k, _):
        pltpu.make_async_copy(zbuf, acc.at[ziota.at[0]], zsem).wait()
        return _

      lax.fori_loop(0, nrows // zrows, zero_issue, None)
      lax.fori_loop(0, nrows // zrows, zero_drain, None)
      plsc.subcore_barrier()

      @pl.when(nq > 0)
      def _edge_phase():
        for b in range(depth):
          pltpu.async_copy(src_hbm.at[pl.ds(idx_off(b), CHUNK)], idx_s[b], ssem[b])
          pltpu.async_copy(dst_hbm.at[pl.ds(idx_off(b), CHUNK)], idx_d[b], dsem[b])
          pltpu.make_async_copy(src_hbm.at[pl.ds(0, CHUNK)], idx_s[b], ssem[b]).wait()
          pltpu.async_copy(h_hbm.at[idx_s[b]], rows[b], gsem[b])

        def body(t, _):
          for b in range(depth):
            g = t * depth + b
            pltpu.make_async_copy(h_hbm.at[idx_s[b]], rows[b], gsem[b]).wait()
            pltpu.async_copy(
                src_hbm.at[pl.ds(idx_off(g + depth), CHUNK)], idx_s[b], ssem[b])
            pltpu.make_async_copy(dst_hbm.at[pl.ds(0, CHUNK)], idx_d[b], dsem[b]).wait()
            pltpu.sync_copy(rows[b], acc.at[idx_d[b]], add=True)
            pltpu.async_copy(
                dst_hbm.at[pl.ds(idx_off(g + depth), CHUNK)], idx_d[b], dsem[b])
            pltpu.make_async_copy(src_hbm.at[pl.ds(0, CHUNK)], idx_s[b], ssem[b]).wait()
            pltpu.async_copy(h_hbm.at[idx_s[b]], rows[b], gsem[b])
          return _

        lax.fori_loop(0, nq // depth - 1, body, None)
        for b in range(depth):
          pltpu.make_async_copy(h_hbm.at[idx_s[b]], rows[b], gsem[b]).wait()
          pltpu.make_async_copy(dst_hbm.at[pl.ds(0, CHUNK)], idx_d[b], dsem[b]).wait()
          pltpu.sync_copy(rows[b], acc.at[idx_d[b]], add=True)

      plsc.subcore_barrier()

      @pl.when(s < NS - 1)
      def _():
        pltpu.sync_copy(acc.at[pl.ds(base, rpt)], out_hbm.at[c, pl.ds(base, rpt)])

      @pl.when(s == NS - 1)
      def _():
        pltpu.sync_copy(acc.at[pl.ds(base, last)], out_hbm.at[c, pl.ds(base, last)])

  return prop_kernel


def _matmul(x, wcat, bcat, block_m=1000):
  n, k = x.shape
  dout = wcat.shape[1]

  def body(x_ref, w_ref, b_ref, o_ref):
    o_ref[...] = (
        jnp.dot(x_ref[...], w_ref[...], preferred_element_type=jnp.float32)
        + b_ref[...]
    )

  return pl.pallas_call(
      body,
      grid=(n // block_m,),
      in_specs=[
          pl.BlockSpec((block_m, k), lambda i: (i, 0)),
          pl.BlockSpec((k, dout), lambda i: (0, 0)),
          pl.BlockSpec((1, dout), lambda i: (0, 0)),
      ],
      out_specs=pl.BlockSpec((block_m, dout), lambda i: (i, 0)),
      out_shape=jax.ShapeDtypeStruct((n, dout), jnp.float32),
  )(x, wcat, bcat)


def _norm_scale(degp0, degp1, t1, t2, block_m=1000):
  n, d = t1.shape

  def body(d0_ref, d1_ref, t1_ref, t2_ref, nv_ref, o1_ref, o2_ref):
    deg = d0_ref[...] + d1_ref[...]
    degc = jnp.max(deg, axis=1, keepdims=True)
    nv = lax.rsqrt(jnp.maximum(degc, 1.0))
    nvb = jnp.broadcast_to(nv, (block_m, d))
    nv_ref[...] = nvb
    o1_ref[...] = t1_ref[...] * nvb
    o2_ref[...] = t2_ref[...] * nvb

  return pl.pallas_call(
      body,
      grid=(n // block_m,),
      in_specs=[
          pl.BlockSpec((block_m, d), lambda i: (i, 0)),
          pl.BlockSpec((block_m, d), lambda i: (i, 0)),
          pl.BlockSpec((block_m, d), lambda i: (i, 0)),
          pl.BlockSpec((block_m, d), lambda i: (i, 0)),
      ],
      out_specs=[pl.BlockSpec((block_m, d), lambda i: (i, 0))] * 3,
      out_shape=[jax.ShapeDtypeStruct((n, d), jnp.float32)] * 3,
  )(degp0, degp1, t1, t2)


def _combine(parts, nvec, npow, block_m=1000):
  _, n, d = parts.shape

  def body(p_ref, nv_ref, o_ref):
    nv = nv_ref[...]
    r = p_ref[0] + p_ref[1]
    for _ in range(npow):
      r = r * nv
    o_ref[...] = r

  return pl.pallas_call(
      body,
      grid=(n // block_m,),
      in_specs=[
          pl.BlockSpec((2, block_m, d), lambda i: (0, i, 0)),
          pl.BlockSpec((block_m, d), lambda i: (i, 0)),
      ],
      out_specs=pl.BlockSpec((block_m, d), lambda i: (i, 0)),
      out_shape=jax.ShapeDtypeStruct((n, d), jnp.float32),
  )(parts, nvec)


def kernel(x, edge_index, W0, b0, W1, b1, W2, b2):
  n, d = x.shape
  e = edge_index.shape[1]
  ncw = -(-e // (NW * CHUNK))
  nch = -(-ncw // 8) * 8
  e_pad = nch * NW * CHUNK
  src = jnp.concatenate(
      [edge_index[0], jnp.zeros((e_pad - e,), jnp.int32)])
  dst = jnp.concatenate(
      [edge_index[1], jnp.full((e_pad - e,), n, jnp.int32)])

  degp = _make_deg_kernel(n, e_pad)(dst.reshape(-1, CHUNK))

  wcat = jnp.concatenate([W0.T, W1.T, W2.T], axis=1)
  bcat = jnp.concatenate([b0, b1, b2]).reshape(1, -1)
  y_all = _matmul(x, wcat, bcat)
  y0, t1, t2 = y_all[:, :d], y_all[:, d:2 * d], y_all[:, 2 * d:]

  nvec, t1n, t2n = _norm_scale(degp[0], degp[1], t1, t2)

  qc = (152, 8) if nch == 80 else None

  p1, p2 = _make_prop_kernel(n, d, e_pad, 2, qc=qc)(t1n, t2n, src, dst)
  y1 = _combine(p1, nvec, 1)
  u2n = _combine(p2, nvec, 2)

  (p3,) = _make_prop_kernel(n, d, e_pad, 1, qc=qc)(u2n, src, dst)
  y2 = _combine(p3, nvec, 1)

  return jnp.concatenate([y0, y1, y2], axis=1)


# speedup vs baseline: 1.0111x; 1.0111x over previous
import functools

import jax
import jax.numpy as jnp
from jax import lax
from jax.experimental import pallas as pl
from jax.experimental.pallas import tpu as pltpu
from jax.experimental.pallas import tpu_sc as plsc

NC = 2
NS = 16
NW = NC * NS
CHUNK = 128

_mesh = functools.partial(
    plsc.VectorSubcoreMesh, core_axis_name="c", subcore_axis_name="s"
)


def _worker_ids():
  c = lax.axis_index("c")
  s = lax.axis_index("s")
  return c, s, c * NS + s


def _row_partition(n):
  zr = 80
  rpt = -(-n // (NS * zr)) * zr
  last = n - rpt * (NS - 1)
  assert last > 0 and last % zr == 0
  return zr, rpt, last


def _make_deg_kernel(n, e_pad):
  nch = e_pad // (NW * CHUNK)
  zr, rpt, last = _row_partition(n)
  n_acc = rpt * NS
  dw = 128

  @functools.partial(
      pl.kernel,
      out_type=jax.ShapeDtypeStruct((NC, n, dw), jnp.float32),
      mesh=_mesh(),
      scratch_types=[
          pltpu.VMEM((nch, CHUNK), jnp.int32),
          pltpu.VMEM((CHUNK, dw), jnp.float32),
          pltpu.VMEM((zr, dw), jnp.float32),
          pltpu.VMEM_SHARED((n_acc, dw), jnp.float32),
          pltpu.SemaphoreType.DMA,
      ],
  )
  def deg_kernel(dst_hbm, out_hbm, idx_d, ones_v, zeros_v, acc, sem):
    c, s, w = _worker_ids()

    cbase = pl.multiple_of(w * nch, nch)
    pltpu.sync_copy(dst_hbm.at[pl.ds(cbase, nch)], idx_d)

    def fill_ones(i, _):
      for j in range(dw // 16):
        ones_v[i, pl.ds(j * 16, 16)] = jnp.full((16,), 1.0, jnp.float32)
      return _

    lax.fori_loop(0, CHUNK, fill_ones, None)

    def fill_zeros(i, _):
      for j in range(dw // 16):
        zeros_v[i, pl.ds(j * 16, 16)] = jnp.zeros((16,), jnp.float32)
      return _

    lax.fori_loop(0, zr, fill_zeros, None)

    base = pl.multiple_of(s * rpt, zr)
    nrows = jnp.where(s == NS - 1, last, rpt)

    def zero_b(b, _):
      pltpu.sync_copy(zeros_v, acc.at[pl.ds(pl.multiple_of(base + b * zr, zr), zr)])
      return _

    lax.fori_loop(0, nrows // zr, zero_b, None)
    plsc.subcore_barrier()

    def body(g, _):
      pltpu.sync_copy(ones_v, acc.at[idx_d.at[g]], add=True)
      return _

    lax.fori_loop(0, nch, body, None)
    plsc.subcore_barrier()

    @pl.when(s < NS - 1)
    def _():
      pltpu.sync_copy(acc.at[pl.ds(base, rpt)], out_hbm.at[c, pl.ds(base, rpt)])

    @pl.when(s == NS - 1)
    def _():
      pltpu.sync_copy(acc.at[pl.ds(base, last)], out_hbm.at[c, pl.ds(base, last)])

  return deg_kernel


def _make_prop_kernel(n, d, e_pad, num_h, depth=2, qc=None):
  nch = e_pad // (NW * CHUNK)
  qc0, qc1 = qc if qc is not None else (nch, nch)
  assert qc0 + qc1 == 2 * nch and qc0 % depth == 0 and qc1 % depth == 0
  zr, rpt, last = _row_partition(n)
  n_acc = rpt * NS
  zrows = 80

  @functools.partial(
      pl.kernel,
      out_type=[jax.ShapeDtypeStruct((NC, n, d), jnp.float32)] * num_h,
      mesh=_mesh(),
      scratch_types=[
          [pltpu.VMEM((CHUNK,), jnp.int32)] * depth,
          [pltpu.VMEM((CHUNK,), jnp.int32)] * depth,
          [pltpu.VMEM((CHUNK, d), jnp.float32)] * depth,
          pltpu.VMEM((zrows, d), jnp.float32),
          pltpu.VMEM((rpt // zrows, zrows), jnp.int32),
          pltpu.VMEM_SHARED((n_acc, d), jnp.float32),
          [pltpu.SemaphoreType.DMA] * depth,
          [pltpu.SemaphoreType.DMA] * depth,
          [pltpu.SemaphoreType.DMA] * depth,
          pltpu.SemaphoreType.DMA,
      ],
  )
  def prop_kernel(*args):
    h_refs = args[:num_h]
    src_hbm, dst_hbm = args[num_h], args[num_h + 1]
    out_refs = args[num_h + 2:num_h + 2 + num_h]
    (idx_s, idx_d, rows, zbuf, ziota, acc,
     gsem, ssem, dsem, zsem) = args[num_h + 2 + num_h:]
    c, s, w = _worker_ids()

    base = pl.multiple_of(s * rpt, zr)
    iota16 = lax.iota(jnp.int32, 16)

    def fill_zeros(i, _):
      for j in range(d // 16):
        zbuf[i, pl.ds(j * 16, 16)] = jnp.zeros((16,), jnp.float32)
      for j in range(zrows // 16):
        ziota[i % (rpt // zrows), pl.ds(j * 16, 16)] = (
            base + (i % (rpt // zrows)) * zrows + j * 16 + iota16)
      return _

    lax.fori_loop(0, max(zrows, rpt // zrows), fill_zeros, None)

    nrows = jnp.where(s == NS - 1, last, rpt)
    nq = jnp.where(c == 0, qc0, qc1)
    cbase = jnp.where(c == 0, s * qc0, NS * qc0 + s * qc1)

    def idx_off(g):
      return pl.multiple_of((cbase + g) * CHUNK, CHUNK)

    for h_hbm, out_hbm in zip(h_refs, out_refs):

      def zero_issue(k, _):
        pltpu.async_copy(zbuf, acc.at[ziota.at[k]], zsem)
        return _

      def zero_drain(k, _):
        pltpu.make_async_copy(zbuf, acc.at[ziota.at[0]], zsem).wait()
        return _

      lax.fori_loop(0, nrows // zrows, zero_issue, None)
      lax.fori_loop(0, nrows // zrows, zero_drain, None)
      plsc.subcore_barrier()

      @pl.when(nq > 0)
      def _edge_phase():
        for b in range(depth):
          pltpu.async_copy(src_hbm.at[pl.ds(idx_off(b), CHUNK)], idx_s[b], ssem[b])
          pltpu.async_copy(dst_hbm.at[pl.ds(idx_off(b), CHUNK)], idx_d[b], dsem[b])
          pltpu.make_async_copy(src_hbm.at[pl.ds(0, CHUNK)], idx_s[b], ssem[b]).wait()
          pltpu.async_copy(h_hbm.at[idx_s[b]], rows[b], gsem[b])

        def body(t, _):
          for b in range(depth):
            g = t * depth + b
            pltpu.make_async_copy(h_hbm.at[idx_s[b]], rows[b], gsem[b]).wait()
            pltpu.async_copy(
                src_hbm.at[pl.ds(idx_off(g + depth), CHUNK)], idx_s[b], ssem[b])
            pltpu.make_async_copy(dst_hbm.at[pl.ds(0, CHUNK)], idx_d[b], dsem[b]).wait()
            pltpu.sync_copy(rows[b], acc.at[idx_d[b]], add=True)
            pltpu.async_copy(
                dst_hbm.at[pl.ds(idx_off(g + depth), CHUNK)], idx_d[b], dsem[b])
            pltpu.make_async_copy(src_hbm.at[pl.ds(0, CHUNK)], idx_s[b], ssem[b]).wait()
            pltpu.async_copy(h_hbm.at[idx_s[b]], rows[b], gsem[b])
          return _

        lax.fori_loop(0, nq // depth - 1, body, None)
        for b in range(depth):
          pltpu.make_async_copy(h_hbm.at[idx_s[b]], rows[b], gsem[b]).wait()
          pltpu.make_async_copy(dst_hbm.at[pl.ds(0, CHUNK)], idx_d[b], dsem[b]).wait()
          pltpu.sync_copy(rows[b], acc.at[idx_d[b]], add=True)

      plsc.subcore_barrier()

      @pl.when(s < NS - 1)
      def _():
        pltpu.sync_copy(acc.at[pl.ds(base, rpt)], out_hbm.at[c, pl.ds(base, rpt)])

      @pl.when(s == NS - 1)
      def _():
        pltpu.sync_copy(acc.at[pl.ds(base, last)], out_hbm.at[c, pl.ds(base, last)])

  return prop_kernel


def _matmul(x, wcat, bcat, block_m=1000):
  n, k = x.shape
  dout = wcat.shape[1]

  def body(x_ref, w_ref, b_ref, o_ref):
    o_ref[...] = (
        jnp.dot(x_ref[...], w_ref[...], preferred_element_type=jnp.float32)
        + b_ref[...]
    )

  return pl.pallas_call(
      body,
      grid=(n // block_m,),
      in_specs=[
          pl.BlockSpec((block_m, k), lambda i: (i, 0)),
          pl.BlockSpec((k, dout), lambda i: (0, 0)),
          pl.BlockSpec((1, dout), lambda i: (0, 0)),
      ],
      out_specs=pl.BlockSpec((block_m, dout), lambda i: (i, 0)),
      out_shape=jax.ShapeDtypeStruct((n, dout), jnp.float32),
  )(x, wcat, bcat)


def _norm_scale(degp0, degp1, t1, t2, block_m=1000):
  n, d = t1.shape

  def body(d0_ref, d1_ref, t1_ref, t2_ref, nv_ref, o1_ref, o2_ref):
    deg = d0_ref[...] + d1_ref[...]
    degc = jnp.max(deg, axis=1, keepdims=True)
    nv = lax.rsqrt(jnp.maximum(degc, 1.0))
    nvb = jnp.broadcast_to(nv, (block_m, d))
    nv_ref[...] = nvb
    o1_ref[...] = t1_ref[...] * nvb
    o2_ref[...] = t2_ref[...] * nvb

  return pl.pallas_call(
      body,
      grid=(n // block_m,),
      in_specs=[
          pl.BlockSpec((block_m, d), lambda i: (i, 0)),
          pl.BlockSpec((block_m, d), lambda i: (i, 0)),
          pl.BlockSpec((block_m, d), lambda i: (i, 0)),
          pl.BlockSpec((block_m, d), lambda i: (i, 0)),
      ],
      out_specs=[pl.BlockSpec((block_m, d), lambda i: (i, 0))] * 3,
      out_shape=[jax.ShapeDtypeStruct((n, d), jnp.float32)] * 3,
  )(degp0, degp1, t1, t2)


def _combine(parts, nvec, npow, block_m=1000):
  _, n, d = parts.shape

  def body(p_ref, nv_ref, o_ref):
    nv = nv_ref[...]
    r = p_ref[0] + p_ref[1]
    for _ in range(npow):
      r = r * nv
    o_ref[...] = r

  return pl.pallas_call(
      body,
      grid=(n // block_m,),
      in_specs=[
          pl.BlockSpec((2, block_m, d), lambda i: (0, i, 0)),
          pl.BlockSpec((block_m, d), lambda i: (i, 0)),
      ],
      out_specs=pl.BlockSpec((block_m, d), lambda i: (i, 0)),
      out_shape=jax.ShapeDtypeStruct((n, d), jnp.float32),
  )(parts, nvec)


def kernel(x, edge_index, W0, b0, W1, b1, W2, b2):
  n, d = x.shape
  e = edge_index.shape[1]
  ncw = -(-e // (NW * CHUNK))
  nch = -(-ncw // 8) * 8
  e_pad = nch * NW * CHUNK
  src = jnp.concatenate(
      [edge_index[0], jnp.zeros((e_pad - e,), jnp.int32)])
  dst = jnp.concatenate(
      [edge_index[1], jnp.full((e_pad - e,), n, jnp.int32)])

  degp = _make_deg_kernel(n, e_pad)(dst.reshape(-1, CHUNK))

  wcat = jnp.concatenate([W0.T, W1.T, W2.T], axis=1)
  bcat = jnp.concatenate([b0, b1, b2]).reshape(1, -1)
  y_all = _matmul(x, wcat, bcat)
  y0, t1, t2 = y_all[:, :d], y_all[:, d:2 * d], y_all[:, 2 * d:]

  nvec, t1n, t2n = _norm_scale(degp[0], degp[1], t1, t2)

  qc = (148, 12) if nch == 80 else None

  p1, p2 = _make_prop_kernel(n, d, e_pad, 2, qc=qc)(t1n, t2n, src, dst)
  y1 = _combine(p1, nvec, 1)
  u2n = _combine(p2, nvec, 2)

  (p3,) = _make_prop_kernel(n, d, e_pad, 1, qc=qc)(u2n, src, dst)
  y2 = _combine(p3, nvec, 1)

  return jnp.concatenate([y0, y1, y2], axis=1)
